# baseline (speedup 1.0000x reference)
import functools
import math

import jax
import jax.numpy as jnp
from jax import lax
from jax.experimental import pallas as pl
from jax.experimental.pallas import tpu as pltpu
from jax.experimental.pallas import tpu_sc as plsc

_VOCAB = 100000
_D = 2048
_BATCH = 2
_S = 4096
_NIDS = _BATCH * _S
_SCALE = float(_D) ** 0.5
_NEG_INF = -2.3819763e+38
_HEAD = 256
_NF = _HEAD // 2
_THETA = 10000.0

_NC = 2
_NS = 16
_L = 16
_NW = _NC * _NS
_BPW = _NIDS // _NW
_CH = 8
_NCH = _BPW // _CH
_NBUF = 4
_VPR = _D // _L


def _make_sc_gather():
  mesh = plsc.VectorSubcoreMesh(core_axis_name="c", subcore_axis_name="s")

  @functools.partial(
      pl.kernel,
      out_type=jax.ShapeDtypeStruct((_NIDS, _D), jnp.float32),
      mesh=mesh,
      scratch_types=[
          pltpu.VMEM((_BPW,), jnp.int32),
      ]
      + [pltpu.VMEM((_CH, _D), jnp.float32)] * _NBUF
      + [pltpu.SemaphoreType.DMA] * (2 * _NBUF),
  )
  def gather_kernel(weight_hbm, ids_hbm, out_hbm, idx_v, *bufs_sems):
    bufs = bufs_sems[:_NBUF]
    gsems = bufs_sems[_NBUF:2 * _NBUF]
    osems = bufs_sems[2 * _NBUF:]
    wid = lax.axis_index("s") * _NC + lax.axis_index("c")
    base = wid * _BPW
    pltpu.sync_copy(ids_hbm.at[pl.ds(base, _BPW)], idx_v)

    def start_gather(ch):
      b = ch % _NBUF
      pltpu.async_copy(
          weight_hbm.at[idx_v.at[pl.ds(ch * _CH, _CH)]], bufs[b], gsems[b])

    def wait_gather(ch):
      b = ch % _NBUF
      pltpu.make_async_copy(
          weight_hbm.at[idx_v.at[pl.ds(ch * _CH, _CH)]], bufs[b],
          gsems[b]).wait()

    def start_out(ch):
      b = ch % _NBUF
      pltpu.async_copy(bufs[b], out_hbm.at[pl.ds(base + ch * _CH, _CH)],
                       osems[b])

    def wait_out(ch):
      b = ch % _NBUF
      pltpu.make_async_copy(bufs[b], out_hbm.at[pl.ds(base + ch * _CH, _CH)],
                            osems[b]).wait()

    def scale(ch):
      buf = bufs[ch % _NBUF]

      @pl.loop(0, _CH)
      def _(r):

        @pl.loop(0, _VPR, unroll=16)
        def _(c):
          buf[r, pl.ds(c * _L, _L)] = buf[r, pl.ds(c * _L, _L)] * _SCALE

    start_gather(0)
    start_gather(1)
    for g in range(_NCH):
      if g + 2 < _NCH:
        if g - 2 >= 0:
          wait_out(g - 2)
        start_gather(g + 2)
      wait_gather(g)
      scale(g)
      start_out(g)
    for ch in range(_NCH - 4, _NCH):
      wait_out(ch)

  return gather_kernel


_sc_gather_cache = []


def _sc_gather(weight, ids):
  if not _sc_gather_cache:
    _sc_gather_cache.append(_make_sc_gather())
  return _sc_gather_cache[0](weight, ids)

_RB = 256


def _mask_freqs_body(mask_ref, cs_ref):
  i = pl.program_id(0)
  rows = i * _RB + lax.broadcasted_iota(jnp.int32, (_RB, _S), 0)
  cols = lax.broadcasted_iota(jnp.int32, (_RB, _S), 1)
  mask_ref[0, 0, :, :] = jnp.where(cols > rows, _NEG_INF, 0.0)
  t = (i * _RB + lax.broadcasted_iota(jnp.int32, (_RB, 2 * _NF), 0)).astype(
      jnp.float32)
  j2 = lax.broadcasted_iota(jnp.int32, (_RB, 2 * _NF), 1)
  k = (j2 // 2).astype(jnp.float32)
  inv_freq = jnp.exp(k * jnp.float32(-math.log(_THETA) / _NF))
  angle = t * inv_freq
  cs_ref[...] = jnp.where(j2 % 2 == 0, jnp.cos(angle), jnp.sin(angle))


_mask_freqs = pl.pallas_call(
    _mask_freqs_body,
    grid=(_S // _RB,),
    out_specs=[
        pl.BlockSpec((1, 1, _RB, _S), lambda i: (0, 0, i, 0)),
        pl.BlockSpec((_RB, 2 * _NF), lambda i: (i, 0)),
    ],
    out_shape=[
        jax.ShapeDtypeStruct((1, 1, _S, _S), jnp.float32),
        jax.ShapeDtypeStruct((_S, 2 * _NF), jnp.float32),
    ],
)


def kernel(input_ids, labels, weight):
  ids = input_ids.reshape(_NIDS)
  hidden = _sc_gather(weight, ids)
  mask, cs = _mask_freqs()
  freqs = cs.view(jnp.complex64)
  return (hidden.reshape(_BATCH, _S, _D), freqs, mask, labels)


# speedup vs baseline: 2.3018x; 2.3018x over previous
import functools
import math

import jax
import jax.numpy as jnp
from jax import lax
from jax.experimental import pallas as pl
from jax.experimental.pallas import tpu as pltpu
from jax.experimental.pallas import tpu_sc as plsc

_VOCAB = 100000
_D = 2048
_BATCH = 2
_S = 4096
_NIDS = _BATCH * _S
_SCALE = float(_D) ** 0.5
_NEG_INF = -2.3819763e+38
_HEAD = 256
_NF = _HEAD // 2
_THETA = 10000.0

_NC = 2
_NS = 16
_L = 16
_NW = _NC * _NS
_BPW = _NIDS // _NW
_CH = 8
_NCH = _BPW // _CH
_NBUF = 4
_VPR = _D // _L


def _make_sc_gather():
  mesh = plsc.VectorSubcoreMesh(core_axis_name="c", subcore_axis_name="s")

  @functools.partial(
      pl.kernel,
      out_type=jax.ShapeDtypeStruct((_NIDS, _D), jnp.float32),
      mesh=mesh,
      scratch_types=[
          pltpu.VMEM((_BPW,), jnp.int32),
      ]
      + [pltpu.VMEM((_CH, _D), jnp.float32)] * _NBUF
      + [pltpu.SemaphoreType.DMA] * (2 * _NBUF),
  )
  def gather_kernel(weight_hbm, ids_hbm, out_hbm, idx_v, *bufs_sems):
    bufs = bufs_sems[:_NBUF]
    gsems = bufs_sems[_NBUF:2 * _NBUF]
    osems = bufs_sems[2 * _NBUF:]
    wid = lax.axis_index("s") * _NC + lax.axis_index("c")
    base = wid * _BPW
    pltpu.sync_copy(ids_hbm.at[pl.ds(base, _BPW)], idx_v)

    def start_gather(ch):
      b = ch % _NBUF
      pltpu.async_copy(
          weight_hbm.at[idx_v.at[pl.ds(ch * _CH, _CH)]], bufs[b], gsems[b])

    def wait_gather(ch):
      b = ch % _NBUF
      pltpu.make_async_copy(
          weight_hbm.at[idx_v.at[pl.ds(ch * _CH, _CH)]], bufs[b],
          gsems[b]).wait()

    def start_out(ch):
      b = ch % _NBUF
      pltpu.async_copy(bufs[b], out_hbm.at[pl.ds(base + ch * _CH, _CH)],
                       osems[b])

    def wait_out(ch):
      b = ch % _NBUF
      pltpu.make_async_copy(bufs[b], out_hbm.at[pl.ds(base + ch * _CH, _CH)],
                            osems[b]).wait()

    def scale(ch):
      buf = bufs[ch % _NBUF]

      @pl.loop(0, _CH)
      def _(r):

        @pl.loop(0, _VPR, unroll=16)
        def _(c):
          buf[r, pl.ds(c * _L, _L)] = buf[r, pl.ds(c * _L, _L)] * _SCALE

    start_gather(0)
    start_gather(1)
    for g in range(_NCH):
      if g + 2 < _NCH:
        if g - 2 >= 0:
          wait_out(g - 2)
        start_gather(g + 2)
      wait_gather(g)
      scale(g)
      start_out(g)
    for ch in range(_NCH - 4, _NCH):
      wait_out(ch)

  return gather_kernel


_sc_gather_cache = []


def _sc_gather(weight, ids):
  if not _sc_gather_cache:
    _sc_gather_cache.append(_make_sc_gather())
  return _sc_gather_cache[0](weight, ids)

_RB = 256


def _mask_freqs_body(mask_ref, re_ref, im_ref):
  i = pl.program_id(0)
  rows = i * _RB + lax.broadcasted_iota(jnp.int32, (_RB, _S), 0)
  cols = lax.broadcasted_iota(jnp.int32, (_RB, _S), 1)
  mask_ref[0, 0, :, :] = jnp.where(cols > rows, _NEG_INF, 0.0)
  t = (i * _RB + lax.broadcasted_iota(jnp.int32, (_RB, _NF), 0)).astype(
      jnp.float32)
  k = lax.broadcasted_iota(jnp.int32, (_RB, _NF), 1).astype(jnp.float32)
  inv_freq = jnp.exp(k * jnp.float32(-math.log(_THETA) / _NF))
  angle = t * inv_freq
  re_ref[...] = jnp.cos(angle)
  im_ref[...] = jnp.sin(angle)


_mask_freqs = pl.pallas_call(
    _mask_freqs_body,
    grid=(_S // _RB,),
    out_specs=[
        pl.BlockSpec((1, 1, _RB, _S), lambda i: (0, 0, i, 0)),
        pl.BlockSpec((_RB, _NF), lambda i: (i, 0)),
        pl.BlockSpec((_RB, _NF), lambda i: (i, 0)),
    ],
    out_shape=[
        jax.ShapeDtypeStruct((1, 1, _S, _S), jnp.float32),
        jax.ShapeDtypeStruct((_S, _NF), jnp.float32),
        jax.ShapeDtypeStruct((_S, _NF), jnp.float32),
    ],
)


def kernel(input_ids, labels, weight):
  ids = input_ids.reshape(_NIDS)
  hidden = _sc_gather(weight, ids)
  mask, re_, im_ = _mask_freqs()
  freqs = re_.astype(jnp.complex64) + im_.astype(jnp.complex64) * (1j)
  return (hidden.reshape(_BATCH, _S, _D), freqs, mask, labels)


# speedup vs baseline: 2.6337x; 1.1442x over previous
import functools
import math

import jax
import jax.numpy as jnp
from jax import lax
from jax.experimental import pallas as pl
from jax.experimental.pallas import tpu as pltpu
from jax.experimental.pallas import tpu_sc as plsc

_VOCAB = 100000
_D = 2048
_BATCH = 2
_S = 4096
_NIDS = _BATCH * _S
_SCALE = float(_D) ** 0.5
_NEG_INF = -2.3819763e+38
_HEAD = 256
_NF = _HEAD // 2
_THETA = 10000.0

_NC = 2
_NS = 16
_L = 16
_NW = _NC * _NS
_BPW = _NIDS // _NW
_CH = 8
_NCH = _BPW // _CH
_NBUF = 4
_VPR = _D // _L


def _make_sc_gather():
  mesh = plsc.VectorSubcoreMesh(core_axis_name="c", subcore_axis_name="s")

  @functools.partial(
      pl.kernel,
      out_type=jax.ShapeDtypeStruct((_NIDS, _D), jnp.float32),
      mesh=mesh,
      scratch_types=[
          pltpu.VMEM((_BPW,), jnp.int32),
      ]
      + [pltpu.VMEM((_CH, _D), jnp.float32)] * _NBUF
      + [pltpu.SemaphoreType.DMA] * (2 * _NBUF),
  )
  def gather_kernel(weight_hbm, ids_hbm, out_hbm, idx_v, *bufs_sems):
    bufs = bufs_sems[:_NBUF]
    gsems = bufs_sems[_NBUF:2 * _NBUF]
    osems = bufs_sems[2 * _NBUF:]
    wid = lax.axis_index("s") * _NC + lax.axis_index("c")
    base = wid * _BPW
    pltpu.sync_copy(ids_hbm.at[pl.ds(base, _BPW)], idx_v)

    def start_gather(ch):
      b = ch % _NBUF
      pltpu.async_copy(
          weight_hbm.at[idx_v.at[pl.ds(ch * _CH, _CH)]], bufs[b], gsems[b])

    def wait_gather(ch):
      b = ch % _NBUF
      pltpu.make_async_copy(
          weight_hbm.at[idx_v.at[pl.ds(ch * _CH, _CH)]], bufs[b],
          gsems[b]).wait()

    def start_out(ch):
      b = ch % _NBUF
      pltpu.async_copy(bufs[b], out_hbm.at[pl.ds(base + ch * _CH, _CH)],
                       osems[b])

    def wait_out(ch):
      b = ch % _NBUF
      pltpu.make_async_copy(bufs[b], out_hbm.at[pl.ds(base + ch * _CH, _CH)],
                            osems[b]).wait()

    def scale(ch):
      buf = bufs[ch % _NBUF]

      @pl.loop(0, _CH)
      def _(r):

        @pl.loop(0, _VPR, unroll=16)
        def _(c):
          buf[r, pl.ds(c * _L, _L)] = buf[r, pl.ds(c * _L, _L)] * _SCALE

    start_gather(0)
    start_gather(1)
    for g in range(_NCH):
      if g + 2 < _NCH:
        if g - 2 >= 0:
          wait_out(g - 2)
        start_gather(g + 2)
      wait_gather(g)
      scale(g)
      start_out(g)
    for ch in range(_NCH - 4, _NCH):
      wait_out(ch)

  return gather_kernel


_sc_gather_cache = []


def _sc_gather(weight, ids):
  if not _sc_gather_cache:
    _sc_gather_cache.append(_make_sc_gather())
  return _sc_gather_cache[0](weight, ids)

_RB = 256


def _mask_body(mask_ref):
  i = pl.program_id(0)
  rows = i * _RB + lax.broadcasted_iota(jnp.int32, (_RB, _S), 0)
  cols = lax.broadcasted_iota(jnp.int32, (_RB, _S), 1)
  mask_ref[0, 0, :, :] = jnp.where(cols > rows, _NEG_INF, 0.0)


_mask = pl.pallas_call(
    _mask_body,
    grid=(_S // _RB,),
    out_specs=pl.BlockSpec((1, 1, _RB, _S), lambda i: (0, 0, i, 0)),
    out_shape=jax.ShapeDtypeStruct((1, 1, _S, _S), jnp.float32),
)


def kernel(input_ids, labels, weight):
  ids = input_ids.reshape(_NIDS)
  hidden = _sc_gather(weight, ids)
  mask = _mask()
  inv_freq = 1.0 / (_THETA ** (
      jnp.arange(0, _HEAD, 2, dtype=jnp.float32)[: _NF] / _HEAD))
  t = jnp.arange(_S, dtype=jnp.float32)
  freqs = jnp.exp(1j * jnp.outer(t, inv_freq).astype(jnp.complex64))
  return (hidden.reshape(_BATCH, _S, _D), freqs, mask, labels)


# speedup vs baseline: 2.6489x; 1.0058x over previous
import functools
import math

import jax
import jax.numpy as jnp
from jax import lax
from jax.experimental import pallas as pl
from jax.experimental.pallas import tpu as pltpu
from jax.experimental.pallas import tpu_sc as plsc

_VOCAB = 100000
_D = 2048
_BATCH = 2
_S = 4096
_NIDS = _BATCH * _S
_SCALE = float(_D) ** 0.5
_NEG_INF = -2.3819763e+38
_HEAD = 256
_NF = _HEAD // 2
_THETA = 10000.0

_NC = 2
_NS = 16
_L = 16
_NW = _NC * _NS
_BPW = _NIDS // _NW
_CH = 8
_NCH = _BPW // _CH
_NBUF = 4
_VPR = _D // _L


def _make_sc_gather():
  mesh = plsc.VectorSubcoreMesh(core_axis_name="c", subcore_axis_name="s")

  @functools.partial(
      pl.kernel,
      out_type=jax.ShapeDtypeStruct((_NIDS, _D), jnp.float32),
      mesh=mesh,
      scratch_types=[
          pltpu.VMEM((_BPW,), jnp.int32),
      ]
      + [pltpu.VMEM((_CH, _D), jnp.float32)] * _NBUF
      + [pltpu.SemaphoreType.DMA] * (2 * _NBUF),
  )
  def gather_kernel(weight_hbm, ids_hbm, out_hbm, idx_v, *bufs_sems):
    bufs = bufs_sems[:_NBUF]
    gsems = bufs_sems[_NBUF:2 * _NBUF]
    osems = bufs_sems[2 * _NBUF:]
    wid = lax.axis_index("s") * _NC + lax.axis_index("c")
    base = wid * _BPW
    row = wid // (_S // _BPW)
    col = (wid % (_S // _BPW)) * _BPW
    pltpu.sync_copy(ids_hbm.at[row, pl.ds(col, _BPW)], idx_v)

    def start_gather(ch):
      b = ch % _NBUF
      pltpu.async_copy(
          weight_hbm.at[idx_v.at[pl.ds(ch * _CH, _CH)]], bufs[b], gsems[b])

    def wait_gather(ch):
      b = ch % _NBUF
      pltpu.make_async_copy(
          weight_hbm.at[idx_v.at[pl.ds(ch * _CH, _CH)]], bufs[b],
          gsems[b]).wait()

    def start_out(ch):
      b = ch % _NBUF
      pltpu.async_copy(bufs[b], out_hbm.at[pl.ds(base + ch * _CH, _CH)],
                       osems[b])

    def wait_out(ch):
      b = ch % _NBUF
      pltpu.make_async_copy(bufs[b], out_hbm.at[pl.ds(base + ch * _CH, _CH)],
                            osems[b]).wait()

    def scale(ch):
      buf = bufs[ch % _NBUF]

      @pl.loop(0, _CH)
      def _(r):

        @pl.loop(0, _VPR, unroll=16)
        def _(c):
          buf[r, pl.ds(c * _L, _L)] = buf[r, pl.ds(c * _L, _L)] * _SCALE

    start_gather(0)
    start_gather(1)
    for g in range(_NCH):
      if g + 2 < _NCH:
        if g - 2 >= 0:
          wait_out(g - 2)
        start_gather(g + 2)
      wait_gather(g)
      scale(g)
      start_out(g)
    for ch in range(_NCH - 4, _NCH):
      wait_out(ch)

  return gather_kernel


_sc_gather_cache = []


def _sc_gather(weight, ids):
  if not _sc_gather_cache:
    _sc_gather_cache.append(_make_sc_gather())
  return _sc_gather_cache[0](weight, ids)

_RB = 256


def _mask_body(mask_ref):
  i = pl.program_id(0)
  rows = i * _RB + lax.broadcasted_iota(jnp.int32, (_RB, _S), 0)
  cols = lax.broadcasted_iota(jnp.int32, (_RB, _S), 1)
  mask_ref[0, 0, :, :] = jnp.where(cols > rows, _NEG_INF, 0.0)


_mask = pl.pallas_call(
    _mask_body,
    grid=(_S // _RB,),
    out_specs=pl.BlockSpec((1, 1, _RB, _S), lambda i: (0, 0, i, 0)),
    out_shape=jax.ShapeDtypeStruct((1, 1, _S, _S), jnp.float32),
)


def kernel(input_ids, labels, weight):
  hidden = _sc_gather(weight, input_ids)
  mask = _mask()
  inv_freq = 1.0 / (_THETA ** (
      jnp.arange(0, _HEAD, 2, dtype=jnp.float32)[: _NF] / _HEAD))
  t = jnp.arange(_S, dtype=jnp.float32)
  freqs = jnp.exp(1j * jnp.outer(t, inv_freq).astype(jnp.complex64))
  return (hidden.reshape(_BATCH, _S, _D), freqs, mask, labels)
